# baseline (speedup 1.0000x reference)
import functools

import jax
import jax.numpy as jnp
from jax import lax
from jax.experimental import pallas as pl
from jax.experimental.pallas import tpu as pltpu
from jax.experimental.pallas import tpu_sc as plsc

B = 16384
D = 64
NC = 2
NS = 16
NW = NC * NS
BPW = B // NW
CHUNK = 128
NCHUNK = BPW // CHUNK
LANES = 16
VPD = D // LANES


def _sc_embed_lookup(X, embed_table, shared_flat):
    mesh = plsc.VectorSubcoreMesh(core_axis_name="c", subcore_axis_name="s")

    @functools.partial(
        pl.kernel,
        mesh=mesh,
        out_type=jax.ShapeDtypeStruct((B, D), jnp.float32),
        compiler_params=pltpu.CompilerParams(use_tc_tiling_on_sc=False),
        scratch_types=[
            pltpu.VMEM((NCHUNK, CHUNK), jnp.int32),
            pltpu.VMEM((BPW, D), jnp.float32),
            pltpu.VMEM((D,), jnp.float32),
            pltpu.SemaphoreType.DMA,
        ],
    )
    def body(x_hbm, tab_hbm, sh_hbm, out_hbm, idx_v, rows_v, sh_v, sem):
        wid = lax.axis_index("s") * NC + lax.axis_index("c")
        base = wid * BPW

        pltpu.sync_copy(sh_hbm, sh_v)
        for j in range(NCHUNK):
            pltpu.sync_copy(
                x_hbm.at[pl.ds(base + j * CHUNK, CHUNK)], idx_v.at[j]
            )
        copies = [
            pltpu.async_copy(
                tab_hbm.at[idx_v.at[j]],
                rows_v.at[pl.ds(j * CHUNK, CHUNK)],
                sem,
            )
            for j in range(NCHUNK)
        ]
        for c in copies:
            c.wait()

        s_vecs = [sh_v[pl.ds(k * LANES, LANES)] for k in range(VPD)]

        def add_row(i, carry):
            for k in range(VPD):
                plsc.addupdate(rows_v.at[i, pl.ds(k * LANES, LANES)], s_vecs[k])
            return carry

        lax.fori_loop(0, BPW, add_row, 0)

        pltpu.sync_copy(rows_v, out_hbm.at[pl.ds(base, BPW)])

    return body(X, embed_table, shared_flat)


def kernel(X, embed_table, shared_embed):
    return _sc_embed_lookup(X, embed_table, shared_embed.reshape(D))
